# Initial kernel scaffold; baseline (speedup 1.0000x reference)
import jax
import jax.numpy as jnp
from jax.experimental import pallas as pl


def kernel(mem1, mem2, query_emb1, query_emb2, coords, labels):
    raise NotImplementedError("write your pallas kernel here")


# speedup vs baseline: 2.5602x; 2.5602x over previous
import jax
import jax.numpy as jnp
from jax.experimental import pallas as pl
from jax.experimental.pallas import tpu as pltpu

_M = 100000
_D = 256
_B = 256
_POS2 = 100.0
_NEG2 = 2500.0
_INV_TEMP = 1.0 / 0.07
_SHIFT = 15.0
_BM = 2000
_NB = _M // _BM


def _main_kernel(qe1_ref, qe2_ref, qc_ref, qc2_ref, m1_ref, m2_ref, c_ref,
                 loss_ref, o1_ref, o2_ref,
                 s1_ref, s2_ref, p1_ref, p2_ref, n_ref):
    step = pl.program_id(0)

    @pl.when(step == 0)
    def _init():
        zeros = jnp.zeros((1, _B), jnp.float32)
        s1_ref[...] = zeros
        s2_ref[...] = zeros
        p1_ref[...] = zeros
        p2_ref[...] = zeros
        n_ref[...] = zeros

    m1 = m1_ref[...]
    m2 = m2_ref[...]
    o1_ref[...] = m1
    o2_ref[...] = m2

    qe1 = qe1_ref[...]
    qe2 = qe2_ref[...]
    qs1 = qe1 * (_INV_TEMP /
                 (jnp.sqrt(jnp.sum(qe1 * qe1, axis=1, keepdims=True)) + 1e-8))
    qs2 = qe2 * (_INV_TEMP /
                 (jnp.sqrt(jnp.sum(qe2 * qe2, axis=1, keepdims=True)) + 1e-8))

    inv1 = 1.0 / (jnp.sqrt(jnp.sum(m1 * m1, axis=1, keepdims=True)) + 1e-8)
    inv2 = 1.0 / (jnp.sqrt(jnp.sum(m2 * m2, axis=1, keepdims=True)) + 1e-8)

    dims = (((1,), (1,)), ((), ()))
    raw1 = jax.lax.dot_general(m1.astype(jnp.bfloat16), qs1.astype(jnp.bfloat16),
                               dims, preferred_element_type=jnp.float32)
    raw2 = jax.lax.dot_general(m2.astype(jnp.bfloat16), qs2.astype(jnp.bfloat16),
                               dims, preferred_element_type=jnp.float32)
    logits1 = raw1 * inv1
    logits2 = raw2 * inv2

    c = c_ref[...]
    qc = qc_ref[...]
    c2 = jnp.sum(c * c, axis=1, keepdims=True)
    qc2 = qc2_ref[...]
    cross = jax.lax.dot_general(c.astype(jnp.bfloat16), qc.astype(jnp.bfloat16),
                                dims, preferred_element_type=jnp.float32)
    d2 = c2 + (qc2 - 2.0 * cross)
    pos = d2 <= _POS2
    valid = jnp.logical_or(pos, d2 > _NEG2)
    pos_f = pos.astype(jnp.float32)

    t1 = jnp.exp(logits1 - _SHIFT)
    t2 = jnp.exp(logits2 - _SHIFT)
    zero = jnp.float32(0.0)
    s1_ref[...] += jnp.sum(jnp.where(valid, t1, zero), axis=0, keepdims=True)
    s2_ref[...] += jnp.sum(jnp.where(valid, t2, zero), axis=0, keepdims=True)
    p1_ref[...] += jnp.sum(logits1 * pos_f, axis=0, keepdims=True)
    p2_ref[...] += jnp.sum(logits2 * pos_f, axis=0, keepdims=True)
    n_ref[...] += jnp.sum(pos_f, axis=0, keepdims=True)

    @pl.when(step == _NB - 1)
    def _finish():
        lse1 = _SHIFT + jnp.log(s1_ref[...])
        lse2 = _SHIFT + jnp.log(s2_ref[...])
        n = n_ref[...]
        ncl = jnp.maximum(n, 1.0)
        rows = (p1_ref[...] - lse1 * n) / ncl + (p2_ref[...] - lse2 * n) / ncl
        loss_ref[...] = -jnp.sum(rows, axis=1, keepdims=True) / _B


def _scatter_kernel(lab_ref, win_ref, qe1_ref, qe2_ref, b1_ref, b2_ref,
                    o1_ref, o2_ref, sem1, sem2):
    del b1_ref, b2_ref
    for i in range(_B):
        l = lab_ref[i]
        w = win_ref[i]
        pltpu.make_async_copy(qe1_ref.at[pl.ds(w, 1)],
                              o1_ref.at[pl.ds(l, 1)], sem1).start()
        pltpu.make_async_copy(qe2_ref.at[pl.ds(w, 1)],
                              o2_ref.at[pl.ds(l, 1)], sem2).start()
    for _ in range(_B):
        pltpu.make_async_copy(qe1_ref.at[pl.ds(0, 1)],
                              o1_ref.at[pl.ds(0, 1)], sem1).wait()
        pltpu.make_async_copy(qe2_ref.at[pl.ds(0, 1)],
                              o2_ref.at[pl.ds(0, 1)], sem2).wait()


def kernel(mem1, mem2, query_emb1, query_emb2, coords, labels):
    labels = labels.astype(jnp.int32)
    cpad = jnp.pad(coords, ((0, 0), (0, 5)))
    qc = jnp.take(coords, labels, axis=0)
    qcpad = jnp.pad(qc, ((0, 0), (0, 5)))
    qc2row = jnp.sum(qc * qc, axis=1)[None, :]

    loss2d, o1, o2 = pl.pallas_call(
        _main_kernel,
        grid=(_NB,),
        in_specs=[
            pl.BlockSpec((_B, _D), lambda i: (0, 0)),
            pl.BlockSpec((_B, _D), lambda i: (0, 0)),
            pl.BlockSpec((_B, 8), lambda i: (0, 0)),
            pl.BlockSpec((1, _B), lambda i: (0, 0)),
            pl.BlockSpec((_BM, _D), lambda i: (i, 0)),
            pl.BlockSpec((_BM, _D), lambda i: (i, 0)),
            pl.BlockSpec((_BM, 8), lambda i: (i, 0)),
        ],
        out_specs=[
            pl.BlockSpec((1, 1), lambda i: (0, 0)),
            pl.BlockSpec((_BM, _D), lambda i: (i, 0)),
            pl.BlockSpec((_BM, _D), lambda i: (i, 0)),
        ],
        out_shape=[
            jax.ShapeDtypeStruct((1, 1), jnp.float32),
            jax.ShapeDtypeStruct((_M, _D), jnp.float32),
            jax.ShapeDtypeStruct((_M, _D), jnp.float32),
        ],
        scratch_shapes=[pltpu.VMEM((1, _B), jnp.float32) for _ in range(5)],
        compiler_params=pltpu.CompilerParams(
            dimension_semantics=("arbitrary",)),
    )(query_emb1, query_emb2, qcpad, qc2row, mem1, mem2, cpad)

    j = jnp.arange(_B, dtype=jnp.int32)
    eq = labels[:, None] == labels[None, :]
    winner = jnp.max(jnp.where(eq, j[None, :], -1), axis=1).astype(jnp.int32)

    grid_spec = pltpu.PrefetchScalarGridSpec(
        num_scalar_prefetch=2,
        grid=(1,),
        in_specs=[
            pl.BlockSpec(memory_space=pl.ANY),
            pl.BlockSpec(memory_space=pl.ANY),
            pl.BlockSpec(memory_space=pl.ANY),
            pl.BlockSpec(memory_space=pl.ANY),
        ],
        out_specs=[
            pl.BlockSpec(memory_space=pl.ANY),
            pl.BlockSpec(memory_space=pl.ANY),
        ],
        scratch_shapes=[pltpu.SemaphoreType.DMA, pltpu.SemaphoreType.DMA],
    )
    new_mem1, new_mem2 = pl.pallas_call(
        _scatter_kernel,
        grid_spec=grid_spec,
        out_shape=[
            jax.ShapeDtypeStruct((_M, _D), jnp.float32),
            jax.ShapeDtypeStruct((_M, _D), jnp.float32),
        ],
        input_output_aliases={4: 0, 5: 1},
        compiler_params=pltpu.CompilerParams(
            dimension_semantics=("arbitrary",)),
    )(labels, winner, query_emb1, query_emb2, o1, o2)

    return (loss2d[0, 0], new_mem1, new_mem2)


# speedup vs baseline: 2.6911x; 1.0511x over previous
import jax
import jax.numpy as jnp
from jax.experimental import pallas as pl
from jax.experimental.pallas import tpu as pltpu

_M = 100000
_D = 256
_B = 256
_POS2 = 100.0
_NEG2 = 2500.0
_INV_TEMP = 1.0 / 0.07
_SHIFT = 15.0
_BM = 4000
_NB = _M // _BM


def _main_kernel(qe1_ref, qe2_ref, qc_ref, qc2_ref, m1_ref, m2_ref, c_ref,
                 loss_ref, o1_ref, o2_ref,
                 s1_ref, s2_ref, p1_ref, p2_ref, n_ref):
    step = pl.program_id(0)

    @pl.when(step == 0)
    def _init():
        zeros = jnp.zeros((1, _B), jnp.float32)
        s1_ref[...] = zeros
        s2_ref[...] = zeros
        p1_ref[...] = zeros
        p2_ref[...] = zeros
        n_ref[...] = zeros

    m1 = m1_ref[...]
    m2 = m2_ref[...]
    o1_ref[...] = m1
    o2_ref[...] = m2

    qe1 = qe1_ref[...]
    qe2 = qe2_ref[...]
    qs1 = qe1 * (_INV_TEMP /
                 (jnp.sqrt(jnp.sum(qe1 * qe1, axis=1, keepdims=True)) + 1e-8))
    qs2 = qe2 * (_INV_TEMP /
                 (jnp.sqrt(jnp.sum(qe2 * qe2, axis=1, keepdims=True)) + 1e-8))

    inv1 = 1.0 / (jnp.sqrt(jnp.sum(m1 * m1, axis=1, keepdims=True)) + 1e-8)
    inv2 = 1.0 / (jnp.sqrt(jnp.sum(m2 * m2, axis=1, keepdims=True)) + 1e-8)

    dims = (((1,), (1,)), ((), ()))
    raw1 = jax.lax.dot_general(m1.astype(jnp.bfloat16), qs1.astype(jnp.bfloat16),
                               dims, preferred_element_type=jnp.float32)
    raw2 = jax.lax.dot_general(m2.astype(jnp.bfloat16), qs2.astype(jnp.bfloat16),
                               dims, preferred_element_type=jnp.float32)
    logits1 = raw1 * inv1
    logits2 = raw2 * inv2

    c = c_ref[...]
    qc = qc_ref[...]
    c2 = jnp.sum(c * c, axis=1, keepdims=True)
    qc2 = qc2_ref[...]
    cross = jax.lax.dot_general(c.astype(jnp.bfloat16), qc.astype(jnp.bfloat16),
                                dims, preferred_element_type=jnp.float32)
    d2 = c2 + (qc2 - 2.0 * cross)
    pos = d2 <= _POS2
    valid = jnp.logical_or(pos, d2 > _NEG2)
    pos_f = pos.astype(jnp.float32)

    t1 = jnp.exp(logits1 - _SHIFT)
    t2 = jnp.exp(logits2 - _SHIFT)
    zero = jnp.float32(0.0)
    s1_ref[...] += jnp.sum(jnp.where(valid, t1, zero), axis=0, keepdims=True)
    s2_ref[...] += jnp.sum(jnp.where(valid, t2, zero), axis=0, keepdims=True)
    p1_ref[...] += jnp.sum(logits1 * pos_f, axis=0, keepdims=True)
    p2_ref[...] += jnp.sum(logits2 * pos_f, axis=0, keepdims=True)
    n_ref[...] += jnp.sum(pos_f, axis=0, keepdims=True)

    @pl.when(step == _NB - 1)
    def _finish():
        lse1 = _SHIFT + jnp.log(s1_ref[...])
        lse2 = _SHIFT + jnp.log(s2_ref[...])
        n = n_ref[...]
        ncl = jnp.maximum(n, 1.0)
        rows = (p1_ref[...] - lse1 * n) / ncl + (p2_ref[...] - lse2 * n) / ncl
        loss_ref[...] = -jnp.sum(rows, axis=1, keepdims=True) / _B


def _scatter_kernel(lab_ref, win_ref, qe1_ref, qe2_ref, b1_ref, b2_ref,
                    o1_ref, o2_ref, sem1, sem2):
    del b1_ref, b2_ref
    for i in range(_B):
        l = lab_ref[i]
        w = win_ref[i]
        pltpu.make_async_copy(qe1_ref.at[pl.ds(w, 1)],
                              o1_ref.at[pl.ds(l, 1)], sem1).start()
        pltpu.make_async_copy(qe2_ref.at[pl.ds(w, 1)],
                              o2_ref.at[pl.ds(l, 1)], sem2).start()
    for _ in range(_B):
        pltpu.make_async_copy(qe1_ref.at[pl.ds(0, 1)],
                              o1_ref.at[pl.ds(0, 1)], sem1).wait()
        pltpu.make_async_copy(qe2_ref.at[pl.ds(0, 1)],
                              o2_ref.at[pl.ds(0, 1)], sem2).wait()


def kernel(mem1, mem2, query_emb1, query_emb2, coords, labels):
    labels = labels.astype(jnp.int32)
    cpad = jnp.pad(coords, ((0, 0), (0, 5)))
    qc = jnp.take(coords, labels, axis=0)
    qcpad = jnp.pad(qc, ((0, 0), (0, 5)))
    qc2row = jnp.sum(qc * qc, axis=1)[None, :]

    loss2d, o1, o2 = pl.pallas_call(
        _main_kernel,
        grid=(_NB,),
        in_specs=[
            pl.BlockSpec((_B, _D), lambda i: (0, 0)),
            pl.BlockSpec((_B, _D), lambda i: (0, 0)),
            pl.BlockSpec((_B, 8), lambda i: (0, 0)),
            pl.BlockSpec((1, _B), lambda i: (0, 0)),
            pl.BlockSpec((_BM, _D), lambda i: (i, 0)),
            pl.BlockSpec((_BM, _D), lambda i: (i, 0)),
            pl.BlockSpec((_BM, 8), lambda i: (i, 0)),
        ],
        out_specs=[
            pl.BlockSpec((1, 1), lambda i: (0, 0)),
            pl.BlockSpec((_BM, _D), lambda i: (i, 0)),
            pl.BlockSpec((_BM, _D), lambda i: (i, 0)),
        ],
        out_shape=[
            jax.ShapeDtypeStruct((1, 1), jnp.float32),
            jax.ShapeDtypeStruct((_M, _D), jnp.float32),
            jax.ShapeDtypeStruct((_M, _D), jnp.float32),
        ],
        scratch_shapes=[pltpu.VMEM((1, _B), jnp.float32) for _ in range(5)],
        compiler_params=pltpu.CompilerParams(
            dimension_semantics=("arbitrary",)),
    )(query_emb1, query_emb2, qcpad, qc2row, mem1, mem2, cpad)

    j = jnp.arange(_B, dtype=jnp.int32)
    eq = labels[:, None] == labels[None, :]
    winner = jnp.max(jnp.where(eq, j[None, :], -1), axis=1).astype(jnp.int32)

    grid_spec = pltpu.PrefetchScalarGridSpec(
        num_scalar_prefetch=2,
        grid=(1,),
        in_specs=[
            pl.BlockSpec(memory_space=pl.ANY),
            pl.BlockSpec(memory_space=pl.ANY),
            pl.BlockSpec(memory_space=pl.ANY),
            pl.BlockSpec(memory_space=pl.ANY),
        ],
        out_specs=[
            pl.BlockSpec(memory_space=pl.ANY),
            pl.BlockSpec(memory_space=pl.ANY),
        ],
        scratch_shapes=[pltpu.SemaphoreType.DMA, pltpu.SemaphoreType.DMA],
    )
    new_mem1, new_mem2 = pl.pallas_call(
        _scatter_kernel,
        grid_spec=grid_spec,
        out_shape=[
            jax.ShapeDtypeStruct((_M, _D), jnp.float32),
            jax.ShapeDtypeStruct((_M, _D), jnp.float32),
        ],
        input_output_aliases={4: 0, 5: 1},
        compiler_params=pltpu.CompilerParams(
            dimension_semantics=("arbitrary",)),
    )(labels, winner, query_emb1, query_emb2, o1, o2)

    return (loss2d[0, 0], new_mem1, new_mem2)


# speedup vs baseline: 2.7226x; 1.0117x over previous
import jax
import jax.numpy as jnp
from jax.experimental import pallas as pl
from jax.experimental.pallas import tpu as pltpu

_M = 100000
_D = 256
_B = 256
_POS2 = 100.0
_NEG2 = 2500.0
_INV_TEMP = 1.0 / 0.07
_SHIFT = 15.0
_BM = 4000
_NB = _M // _BM


def _main_kernel(qe1_ref, qe2_ref, qc_ref, qc2_ref, m1_ref, m2_ref, c_ref,
                 loss_ref, o1_ref, o2_ref,
                 s1_ref, s2_ref, p_ref, n_ref):
    step = pl.program_id(0)

    @pl.when(step == 0)
    def _init():
        zeros = jnp.zeros((1, _B), jnp.float32)
        s1_ref[...] = zeros
        s2_ref[...] = zeros
        p_ref[...] = zeros
        n_ref[...] = zeros

    m1 = m1_ref[...]
    m2 = m2_ref[...]
    o1_ref[...] = m1
    o2_ref[...] = m2

    qe1 = qe1_ref[...]
    qe2 = qe2_ref[...]
    qs1 = qe1 * (_INV_TEMP /
                 (jnp.sqrt(jnp.sum(qe1 * qe1, axis=1, keepdims=True)) + 1e-8))
    qs2 = qe2 * (_INV_TEMP /
                 (jnp.sqrt(jnp.sum(qe2 * qe2, axis=1, keepdims=True)) + 1e-8))

    dims = (((1,), (1,)), ((), ()))
    m1b = m1.astype(jnp.bfloat16)
    m2b = m2.astype(jnp.bfloat16)
    ones_col = jnp.ones((8, _D), jnp.float32).astype(jnp.bfloat16)
    n2_1 = jax.lax.dot_general(m1b * m1b, ones_col, dims,
                               preferred_element_type=jnp.float32)
    n2_2 = jax.lax.dot_general(m2b * m2b, ones_col, dims,
                               preferred_element_type=jnp.float32)
    inv1 = 1.0 / (jnp.sqrt(n2_1[:, 0:1]) + 1e-8)
    inv2 = 1.0 / (jnp.sqrt(n2_2[:, 0:1]) + 1e-8)

    raw1 = jax.lax.dot_general(m1b, qs1.astype(jnp.bfloat16),
                               dims, preferred_element_type=jnp.float32)
    raw2 = jax.lax.dot_general(m2b, qs2.astype(jnp.bfloat16),
                               dims, preferred_element_type=jnp.float32)
    logits1 = raw1 * inv1
    logits2 = raw2 * inv2

    c = c_ref[...]
    qc = qc_ref[...]
    c2h = 0.5 * jnp.sum(c * c, axis=1, keepdims=True)
    qc2h = qc2_ref[...]
    cross = jax.lax.dot_general(c.astype(jnp.bfloat16), qc.astype(jnp.bfloat16),
                                dims, preferred_element_type=jnp.float32)
    u = c2h + qc2h
    pos_f = (cross >= u - (0.5 * _POS2)).astype(jnp.float32)
    neg_f = (cross < u - (0.5 * _NEG2)).astype(jnp.float32)
    valid_f = pos_f + neg_f

    t1 = jnp.exp(logits1)
    t2 = jnp.exp(logits2)
    s1_ref[...] += jnp.sum(t1 * valid_f, axis=0, keepdims=True)
    s2_ref[...] += jnp.sum(t2 * valid_f, axis=0, keepdims=True)
    p_ref[...] += jnp.sum((logits1 + logits2) * pos_f, axis=0, keepdims=True)
    n_ref[...] += jnp.sum(pos_f, axis=0, keepdims=True)

    @pl.when(step == _NB - 1)
    def _finish():
        lse12 = jnp.log(s1_ref[...]) + jnp.log(s2_ref[...])
        n = n_ref[...]
        ncl = jnp.maximum(n, 1.0)
        rows = (p_ref[...] - lse12 * n) / ncl
        loss_ref[...] = -jnp.sum(rows, axis=1, keepdims=True) / _B


def _scatter_kernel(lab_ref, win_ref, qe1_ref, qe2_ref, b1_ref, b2_ref,
                    o1_ref, o2_ref, sem1, sem2):
    del b1_ref, b2_ref
    for i in range(_B):
        l = lab_ref[i]
        w = win_ref[i]
        pltpu.make_async_copy(qe1_ref.at[pl.ds(w, 1)],
                              o1_ref.at[pl.ds(l, 1)], sem1).start()
        pltpu.make_async_copy(qe2_ref.at[pl.ds(w, 1)],
                              o2_ref.at[pl.ds(l, 1)], sem2).start()
    for _ in range(_B):
        pltpu.make_async_copy(qe1_ref.at[pl.ds(0, 1)],
                              o1_ref.at[pl.ds(0, 1)], sem1).wait()
        pltpu.make_async_copy(qe2_ref.at[pl.ds(0, 1)],
                              o2_ref.at[pl.ds(0, 1)], sem2).wait()


def kernel(mem1, mem2, query_emb1, query_emb2, coords, labels):
    labels = labels.astype(jnp.int32)
    cpad = jnp.pad(coords, ((0, 0), (0, 5)))
    qc = jnp.take(coords, labels, axis=0)
    qcpad = jnp.pad(qc, ((0, 0), (0, 5)))
    qc2row = 0.5 * jnp.sum(qc * qc, axis=1)[None, :]

    loss2d, o1, o2 = pl.pallas_call(
        _main_kernel,
        grid=(_NB,),
        in_specs=[
            pl.BlockSpec((_B, _D), lambda i: (0, 0)),
            pl.BlockSpec((_B, _D), lambda i: (0, 0)),
            pl.BlockSpec((_B, 8), lambda i: (0, 0)),
            pl.BlockSpec((1, _B), lambda i: (0, 0)),
            pl.BlockSpec((_BM, _D), lambda i: (i, 0)),
            pl.BlockSpec((_BM, _D), lambda i: (i, 0)),
            pl.BlockSpec((_BM, 8), lambda i: (i, 0)),
        ],
        out_specs=[
            pl.BlockSpec((1, 1), lambda i: (0, 0)),
            pl.BlockSpec((_BM, _D), lambda i: (i, 0)),
            pl.BlockSpec((_BM, _D), lambda i: (i, 0)),
        ],
        out_shape=[
            jax.ShapeDtypeStruct((1, 1), jnp.float32),
            jax.ShapeDtypeStruct((_M, _D), jnp.float32),
            jax.ShapeDtypeStruct((_M, _D), jnp.float32),
        ],
        scratch_shapes=[pltpu.VMEM((1, _B), jnp.float32) for _ in range(4)],
        compiler_params=pltpu.CompilerParams(
            dimension_semantics=("arbitrary",)),
    )(query_emb1, query_emb2, qcpad, qc2row, mem1, mem2, cpad)

    j = jnp.arange(_B, dtype=jnp.int32)
    eq = labels[:, None] == labels[None, :]
    winner = jnp.max(jnp.where(eq, j[None, :], -1), axis=1).astype(jnp.int32)

    grid_spec = pltpu.PrefetchScalarGridSpec(
        num_scalar_prefetch=2,
        grid=(1,),
        in_specs=[
            pl.BlockSpec(memory_space=pl.ANY),
            pl.BlockSpec(memory_space=pl.ANY),
            pl.BlockSpec(memory_space=pl.ANY),
            pl.BlockSpec(memory_space=pl.ANY),
        ],
        out_specs=[
            pl.BlockSpec(memory_space=pl.ANY),
            pl.BlockSpec(memory_space=pl.ANY),
        ],
        scratch_shapes=[pltpu.SemaphoreType.DMA, pltpu.SemaphoreType.DMA],
    )
    new_mem1, new_mem2 = pl.pallas_call(
        _scatter_kernel,
        grid_spec=grid_spec,
        out_shape=[
            jax.ShapeDtypeStruct((_M, _D), jnp.float32),
            jax.ShapeDtypeStruct((_M, _D), jnp.float32),
        ],
        input_output_aliases={4: 0, 5: 1},
        compiler_params=pltpu.CompilerParams(
            dimension_semantics=("arbitrary",)),
    )(labels, winner, query_emb1, query_emb2, o1, o2)

    return (loss2d[0, 0], new_mem1, new_mem2)


# speedup vs baseline: 2.8826x; 1.0588x over previous
import jax
import jax.numpy as jnp
from jax.experimental import pallas as pl
from jax.experimental.pallas import tpu as pltpu

_M = 100000
_D = 256
_B = 256
_POS2 = 100.0
_NEG2 = 2500.0
_INV_TEMP = 1.0 / 0.07
_SHIFT = 15.0
_BM = 4000
_NB = _M // _BM


def _main_kernel(qe1_ref, qe2_ref, qc_ref, qc2_ref, m1_ref, m2_ref, c_ref,
                 loss_ref, o1_ref, o2_ref,
                 s1_ref, s2_ref, p_ref, n_ref):
    step = pl.program_id(0)

    @pl.when(step == 0)
    def _init():
        zeros = jnp.zeros((1, _B), jnp.float32)
        s1_ref[...] = zeros
        s2_ref[...] = zeros
        p_ref[...] = zeros
        n_ref[...] = zeros

    m1 = m1_ref[...]
    m2 = m2_ref[...]
    o1_ref[...] = m1
    o2_ref[...] = m2

    qe1 = qe1_ref[...]
    qe2 = qe2_ref[...]
    qs1 = qe1 * (_INV_TEMP /
                 (jnp.sqrt(jnp.sum(qe1 * qe1, axis=1, keepdims=True)) + 1e-8))
    qs2 = qe2 * (_INV_TEMP /
                 (jnp.sqrt(jnp.sum(qe2 * qe2, axis=1, keepdims=True)) + 1e-8))

    dims = (((1,), (1,)), ((), ()))
    m1b = m1.astype(jnp.bfloat16)
    m2b = m2.astype(jnp.bfloat16)
    ones_col = jnp.ones((8, _D), jnp.float32).astype(jnp.bfloat16)
    n2_1 = jax.lax.dot_general(m1b * m1b, ones_col, dims,
                               preferred_element_type=jnp.float32)
    n2_2 = jax.lax.dot_general(m2b * m2b, ones_col, dims,
                               preferred_element_type=jnp.float32)
    inv1 = 1.0 / (jnp.sqrt(n2_1[:, 0:1]) + 1e-8)
    inv2 = 1.0 / (jnp.sqrt(n2_2[:, 0:1]) + 1e-8)

    raw1 = jax.lax.dot_general(m1b, qs1.astype(jnp.bfloat16),
                               dims, preferred_element_type=jnp.float32)
    raw2 = jax.lax.dot_general(m2b, qs2.astype(jnp.bfloat16),
                               dims, preferred_element_type=jnp.float32)
    logits1 = raw1 * inv1
    logits2 = raw2 * inv2

    c = c_ref[...]
    qc = qc_ref[...]
    c2h = 0.5 * jnp.sum(c * c, axis=1, keepdims=True)
    qc2h = qc2_ref[...]
    cross = jax.lax.dot_general(c.astype(jnp.bfloat16), qc.astype(jnp.bfloat16),
                                dims, preferred_element_type=jnp.float32)
    u = c2h + qc2h
    pos_f = (cross >= u - (0.5 * _POS2)).astype(jnp.float32)
    neg_f = (cross < u - (0.5 * _NEG2)).astype(jnp.float32)
    valid_f = pos_f + neg_f

    t1 = jnp.exp(logits1)
    t2 = jnp.exp(logits2)
    s1_ref[...] += jnp.sum(t1 * valid_f, axis=0, keepdims=True)
    s2_ref[...] += jnp.sum(t2 * valid_f, axis=0, keepdims=True)
    p_ref[...] += jnp.sum((logits1 + logits2) * pos_f, axis=0, keepdims=True)
    n_ref[...] += jnp.sum(pos_f, axis=0, keepdims=True)

    @pl.when(step == _NB - 1)
    def _finish():
        lse12 = jnp.log(s1_ref[...]) + jnp.log(s2_ref[...])
        n = n_ref[...]
        ncl = jnp.maximum(n, 1.0)
        rows = (p_ref[...] - lse12 * n) / ncl
        loss_ref[...] = -jnp.sum(rows, axis=1, keepdims=True) / _B


def _gather_kernel(lab_ref, c_ref, qc_ref, sem):
    for i in range(_B):
        l = lab_ref[i]
        pltpu.make_async_copy(c_ref.at[pl.ds(l, 1), :],
                              qc_ref.at[pl.ds(i, 1), :], sem).start()
    for _ in range(_B):
        pltpu.make_async_copy(c_ref.at[pl.ds(0, 1), :],
                              qc_ref.at[pl.ds(0, 1), :], sem).wait()


def _scatter_kernel(lab_ref, win_ref, qe1_ref, qe2_ref, b1_ref, b2_ref,
                    o1_ref, o2_ref, sem1, sem2):
    del b1_ref, b2_ref
    for i in range(_B):
        l = lab_ref[i]
        w = win_ref[i]
        pltpu.make_async_copy(qe1_ref.at[pl.ds(w, 1)],
                              o1_ref.at[pl.ds(l, 1)], sem1).start()
        pltpu.make_async_copy(qe2_ref.at[pl.ds(w, 1)],
                              o2_ref.at[pl.ds(l, 1)], sem2).start()
    for _ in range(_B):
        pltpu.make_async_copy(qe1_ref.at[pl.ds(0, 1)],
                              o1_ref.at[pl.ds(0, 1)], sem1).wait()
        pltpu.make_async_copy(qe2_ref.at[pl.ds(0, 1)],
                              o2_ref.at[pl.ds(0, 1)], sem2).wait()


def kernel(mem1, mem2, query_emb1, query_emb2, coords, labels):
    labels = labels.astype(jnp.int32)
    cpad = jnp.pad(coords, ((0, 0), (0, 5)))

    qcpad = pl.pallas_call(
        _gather_kernel,
        grid_spec=pltpu.PrefetchScalarGridSpec(
            num_scalar_prefetch=1,
            grid=(1,),
            in_specs=[pl.BlockSpec(memory_space=pl.ANY)],
            out_specs=pl.BlockSpec((_B, 8), lambda i, lab: (0, 0)),
            scratch_shapes=[pltpu.SemaphoreType.DMA],
        ),
        out_shape=jax.ShapeDtypeStruct((_B, 8), jnp.float32),
        compiler_params=pltpu.CompilerParams(
            dimension_semantics=("arbitrary",)),
    )(labels, cpad)
    qc2row = 0.5 * jnp.sum(qcpad * qcpad, axis=1)[None, :]

    loss2d, o1, o2 = pl.pallas_call(
        _main_kernel,
        grid=(_NB,),
        in_specs=[
            pl.BlockSpec((_B, _D), lambda i: (0, 0)),
            pl.BlockSpec((_B, _D), lambda i: (0, 0)),
            pl.BlockSpec((_B, 8), lambda i: (0, 0)),
            pl.BlockSpec((1, _B), lambda i: (0, 0)),
            pl.BlockSpec((_BM, _D), lambda i: (i, 0)),
            pl.BlockSpec((_BM, _D), lambda i: (i, 0)),
            pl.BlockSpec((_BM, 8), lambda i: (i, 0)),
        ],
        out_specs=[
            pl.BlockSpec((1, 1), lambda i: (0, 0)),
            pl.BlockSpec((_BM, _D), lambda i: (i, 0)),
            pl.BlockSpec((_BM, _D), lambda i: (i, 0)),
        ],
        out_shape=[
            jax.ShapeDtypeStruct((1, 1), jnp.float32),
            jax.ShapeDtypeStruct((_M, _D), jnp.float32),
            jax.ShapeDtypeStruct((_M, _D), jnp.float32),
        ],
        scratch_shapes=[pltpu.VMEM((1, _B), jnp.float32) for _ in range(4)],
        compiler_params=pltpu.CompilerParams(
            dimension_semantics=("arbitrary",)),
    )(query_emb1, query_emb2, qcpad, qc2row, mem1, mem2, cpad)

    j = jnp.arange(_B, dtype=jnp.int32)
    eq = labels[:, None] == labels[None, :]
    winner = jnp.max(jnp.where(eq, j[None, :], -1), axis=1).astype(jnp.int32)

    grid_spec = pltpu.PrefetchScalarGridSpec(
        num_scalar_prefetch=2,
        grid=(1,),
        in_specs=[
            pl.BlockSpec(memory_space=pl.ANY),
            pl.BlockSpec(memory_space=pl.ANY),
            pl.BlockSpec(memory_space=pl.ANY),
            pl.BlockSpec(memory_space=pl.ANY),
        ],
        out_specs=[
            pl.BlockSpec(memory_space=pl.ANY),
            pl.BlockSpec(memory_space=pl.ANY),
        ],
        scratch_shapes=[pltpu.SemaphoreType.DMA, pltpu.SemaphoreType.DMA],
    )
    new_mem1, new_mem2 = pl.pallas_call(
        _scatter_kernel,
        grid_spec=grid_spec,
        out_shape=[
            jax.ShapeDtypeStruct((_M, _D), jnp.float32),
            jax.ShapeDtypeStruct((_M, _D), jnp.float32),
        ],
        input_output_aliases={4: 0, 5: 1},
        compiler_params=pltpu.CompilerParams(
            dimension_semantics=("arbitrary",)),
    )(labels, winner, query_emb1, query_emb2, o1, o2)

    return (loss2d[0, 0], new_mem1, new_mem2)


# speedup vs baseline: 3.3166x; 1.1506x over previous
import jax
import jax.numpy as jnp
from jax.experimental import pallas as pl
from jax.experimental.pallas import tpu as pltpu

_M = 100000
_D = 256
_B = 256
_POS2 = 100.0
_NEG2 = 2500.0
_INV_TEMP = 1.0 / 0.07
_SHIFT = 15.0
_BM = 4000
_NB = _M // _BM


def _main_kernel(qe1_ref, qe2_ref, qc_ref, qc2_ref, m1_ref, m2_ref, c_ref,
                 loss_ref, o1_ref, o2_ref,
                 s1_ref, s2_ref, p_ref, n_ref):
    step = pl.program_id(0)

    @pl.when(step == 0)
    def _init():
        zeros = jnp.zeros((1, _B), jnp.float32)
        s1_ref[...] = zeros
        s2_ref[...] = zeros
        p_ref[...] = zeros
        n_ref[...] = zeros

    m1 = m1_ref[...]
    m2 = m2_ref[...]
    o1_ref[...] = m1
    o2_ref[...] = m2

    qe1 = qe1_ref[...]
    qe2 = qe2_ref[...]
    qs1 = qe1 * (_INV_TEMP /
                 (jnp.sqrt(jnp.sum(qe1 * qe1, axis=1, keepdims=True)) + 1e-8))
    qs2 = qe2 * (_INV_TEMP /
                 (jnp.sqrt(jnp.sum(qe2 * qe2, axis=1, keepdims=True)) + 1e-8))

    dims = (((1,), (1,)), ((), ()))
    m1b = m1.astype(jnp.bfloat16)
    m2b = m2.astype(jnp.bfloat16)
    ones_col = jnp.ones((8, _D), jnp.float32).astype(jnp.bfloat16)
    n2_1 = jax.lax.dot_general(m1b * m1b, ones_col, dims,
                               preferred_element_type=jnp.float32)
    n2_2 = jax.lax.dot_general(m2b * m2b, ones_col, dims,
                               preferred_element_type=jnp.float32)
    inv1 = 1.0 / (jnp.sqrt(n2_1[:, 0:1]) + 1e-8)
    inv2 = 1.0 / (jnp.sqrt(n2_2[:, 0:1]) + 1e-8)

    raw1 = jax.lax.dot_general(m1b, qs1.astype(jnp.bfloat16),
                               dims, preferred_element_type=jnp.float32)
    raw2 = jax.lax.dot_general(m2b, qs2.astype(jnp.bfloat16),
                               dims, preferred_element_type=jnp.float32)
    logits1 = raw1 * inv1
    logits2 = raw2 * inv2

    c = c_ref[...]
    qc = qc_ref[...]
    c2h = 0.5 * jnp.sum(c * c, axis=1, keepdims=True)
    qc2h = qc2_ref[...]
    cross = jax.lax.dot_general(c.astype(jnp.bfloat16), qc.astype(jnp.bfloat16),
                                dims, preferred_element_type=jnp.float32)
    u = c2h + qc2h
    pos_f = (cross >= u - (0.5 * _POS2)).astype(jnp.float32)
    neg_f = (cross < u - (0.5 * _NEG2)).astype(jnp.float32)
    valid_f = pos_f + neg_f

    t1 = jnp.exp(logits1)
    t2 = jnp.exp(logits2)
    s1_ref[...] += jnp.sum(t1 * valid_f, axis=0, keepdims=True)
    s2_ref[...] += jnp.sum(t2 * valid_f, axis=0, keepdims=True)
    p_ref[...] += jnp.sum((logits1 + logits2) * pos_f, axis=0, keepdims=True)
    n_ref[...] += jnp.sum(pos_f, axis=0, keepdims=True)

    @pl.when(step == _NB - 1)
    def _finish():
        lse12 = jnp.log(s1_ref[...]) + jnp.log(s2_ref[...])
        n = n_ref[...]
        ncl = jnp.maximum(n, 1.0)
        rows = (p_ref[...] - lse12 * n) / ncl
        loss_ref[...] = -jnp.sum(rows, axis=1, keepdims=True) / _B


def _gather_kernel(lab_ref, c_ref, qc_ref, sem):
    for i in range(_B):
        l = lab_ref[i]
        pltpu.make_async_copy(c_ref.at[pl.ds(l, 1), :],
                              qc_ref.at[pl.ds(i, 1), :], sem).start()
    for _ in range(_B):
        pltpu.make_async_copy(c_ref.at[pl.ds(0, 1), :],
                              qc_ref.at[pl.ds(0, 1), :], sem).wait()


def _scatter_kernel(lab_ref, win_ref, qe1_ref, qe2_ref, b1_ref, b2_ref,
                    o1_ref, o2_ref, sem1, sem2):
    del b1_ref, b2_ref
    for i in range(_B):
        l = lab_ref[i]
        w = win_ref[i]
        pltpu.make_async_copy(qe1_ref.at[pl.ds(w, 1)],
                              o1_ref.at[pl.ds(l, 1)], sem1).start()
        pltpu.make_async_copy(qe2_ref.at[pl.ds(w, 1)],
                              o2_ref.at[pl.ds(l, 1)], sem2).start()
    for _ in range(_B):
        pltpu.make_async_copy(qe1_ref.at[pl.ds(0, 1)],
                              o1_ref.at[pl.ds(0, 1)], sem1).wait()
        pltpu.make_async_copy(qe2_ref.at[pl.ds(0, 1)],
                              o2_ref.at[pl.ds(0, 1)], sem2).wait()


def kernel(mem1, mem2, query_emb1, query_emb2, coords, labels):
    labels = labels.astype(jnp.int32)

    qc3 = pl.pallas_call(
        _gather_kernel,
        grid_spec=pltpu.PrefetchScalarGridSpec(
            num_scalar_prefetch=1,
            grid=(1,),
            in_specs=[pl.BlockSpec(memory_space=pl.ANY)],
            out_specs=pl.BlockSpec((_B, 3), lambda i, lab: (0, 0)),
            scratch_shapes=[pltpu.SemaphoreType.DMA],
        ),
        out_shape=jax.ShapeDtypeStruct((_B, 3), jnp.float32),
        compiler_params=pltpu.CompilerParams(
            dimension_semantics=("arbitrary",)),
    )(labels, coords)
    qc2row = 0.5 * jnp.sum(qc3 * qc3, axis=1)[None, :]

    loss2d, o1, o2 = pl.pallas_call(
        _main_kernel,
        grid=(_NB,),
        in_specs=[
            pl.BlockSpec((_B, _D), lambda i: (0, 0)),
            pl.BlockSpec((_B, _D), lambda i: (0, 0)),
            pl.BlockSpec((_B, 3), lambda i: (0, 0)),
            pl.BlockSpec((1, _B), lambda i: (0, 0)),
            pl.BlockSpec((_BM, _D), lambda i: (i, 0)),
            pl.BlockSpec((_BM, _D), lambda i: (i, 0)),
            pl.BlockSpec((_BM, 3), lambda i: (i, 0)),
        ],
        out_specs=[
            pl.BlockSpec((1, 1), lambda i: (0, 0)),
            pl.BlockSpec((_BM, _D), lambda i: (i, 0)),
            pl.BlockSpec((_BM, _D), lambda i: (i, 0)),
        ],
        out_shape=[
            jax.ShapeDtypeStruct((1, 1), jnp.float32),
            jax.ShapeDtypeStruct((_M, _D), jnp.float32),
            jax.ShapeDtypeStruct((_M, _D), jnp.float32),
        ],
        scratch_shapes=[pltpu.VMEM((1, _B), jnp.float32) for _ in range(4)],
        compiler_params=pltpu.CompilerParams(
            dimension_semantics=("arbitrary",)),
    )(query_emb1, query_emb2, qc3, qc2row, mem1, mem2, coords)

    j = jnp.arange(_B, dtype=jnp.int32)
    eq = labels[:, None] == labels[None, :]
    winner = jnp.max(jnp.where(eq, j[None, :], -1), axis=1).astype(jnp.int32)

    grid_spec = pltpu.PrefetchScalarGridSpec(
        num_scalar_prefetch=2,
        grid=(1,),
        in_specs=[
            pl.BlockSpec(memory_space=pl.ANY),
            pl.BlockSpec(memory_space=pl.ANY),
            pl.BlockSpec(memory_space=pl.ANY),
            pl.BlockSpec(memory_space=pl.ANY),
        ],
        out_specs=[
            pl.BlockSpec(memory_space=pl.ANY),
            pl.BlockSpec(memory_space=pl.ANY),
        ],
        scratch_shapes=[pltpu.SemaphoreType.DMA, pltpu.SemaphoreType.DMA],
    )
    new_mem1, new_mem2 = pl.pallas_call(
        _scatter_kernel,
        grid_spec=grid_spec,
        out_shape=[
            jax.ShapeDtypeStruct((_M, _D), jnp.float32),
            jax.ShapeDtypeStruct((_M, _D), jnp.float32),
        ],
        input_output_aliases={4: 0, 5: 1},
        compiler_params=pltpu.CompilerParams(
            dimension_semantics=("arbitrary",)),
    )(labels, winner, query_emb1, query_emb2, o1, o2)

    return (loss2d[0, 0], new_mem1, new_mem2)


# speedup vs baseline: 3.3599x; 1.0131x over previous
import jax
import jax.numpy as jnp
from jax.experimental import pallas as pl
from jax.experimental.pallas import tpu as pltpu

_M = 100000
_D = 256
_B = 256
_POS2 = 100.0
_NEG2 = 2500.0
_INV_TEMP = 1.0 / 0.07
_SHIFT = 15.0
_BM = 4000
_NB = _M // _BM


def _main_kernel(qe1_ref, qe2_ref, qc_ref, qc2_ref, m1_ref, m2_ref, c_ref,
                 loss_ref, o1_ref, o2_ref,
                 s1_ref, s2_ref, p_ref, n_ref, qs1_ref, qs2_ref):
    step = pl.program_id(0)

    @pl.when(step == 0)
    def _init():
        zeros = jnp.zeros((1, _B), jnp.float32)
        s1_ref[...] = zeros
        s2_ref[...] = zeros
        p_ref[...] = zeros
        n_ref[...] = zeros

    @pl.when(step == 0)
    def _queries():
        qe1 = qe1_ref[...]
        qe2 = qe2_ref[...]
        qs1 = qe1 * (_INV_TEMP /
                     (jnp.sqrt(jnp.sum(qe1 * qe1, axis=1, keepdims=True)) + 1e-8))
        qs2 = qe2 * (_INV_TEMP /
                     (jnp.sqrt(jnp.sum(qe2 * qe2, axis=1, keepdims=True)) + 1e-8))
        qs1_ref[...] = qs1.astype(jnp.bfloat16)
        qs2_ref[...] = qs2.astype(jnp.bfloat16)

    m1 = m1_ref[...]
    m2 = m2_ref[...]
    o1_ref[...] = m1
    o2_ref[...] = m2

    dims = (((1,), (1,)), ((), ()))
    m1b = m1.astype(jnp.bfloat16)
    m2b = m2.astype(jnp.bfloat16)
    ones_col = jnp.ones((8, _D), jnp.float32).astype(jnp.bfloat16)
    n2_1 = jax.lax.dot_general(m1b * m1b, ones_col, dims,
                               preferred_element_type=jnp.float32)
    n2_2 = jax.lax.dot_general(m2b * m2b, ones_col, dims,
                               preferred_element_type=jnp.float32)
    inv1 = 1.0 / (jnp.sqrt(n2_1[:, 0:1]) + 1e-8)
    inv2 = 1.0 / (jnp.sqrt(n2_2[:, 0:1]) + 1e-8)

    raw1 = jax.lax.dot_general(m1b, qs1_ref[...],
                               dims, preferred_element_type=jnp.float32)
    raw2 = jax.lax.dot_general(m2b, qs2_ref[...],
                               dims, preferred_element_type=jnp.float32)
    logits1 = raw1 * inv1
    logits2 = raw2 * inv2

    c = c_ref[...]
    qc = qc_ref[...]
    c2h = 0.5 * jnp.sum(c * c, axis=1, keepdims=True)
    qc2h = qc2_ref[...]
    cross = jax.lax.dot_general(c.astype(jnp.bfloat16), qc.astype(jnp.bfloat16),
                                dims, preferred_element_type=jnp.float32)
    u = c2h + qc2h
    pos_f = (cross >= u - (0.5 * _POS2)).astype(jnp.float32)
    neg_f = (cross < u - (0.5 * _NEG2)).astype(jnp.float32)
    valid_f = pos_f + neg_f

    t1 = jnp.exp(logits1)
    t2 = jnp.exp(logits2)
    s1_ref[...] += jnp.sum(t1 * valid_f, axis=0, keepdims=True)
    s2_ref[...] += jnp.sum(t2 * valid_f, axis=0, keepdims=True)
    p_ref[...] += jnp.sum((logits1 + logits2) * pos_f, axis=0, keepdims=True)
    n_ref[...] += jnp.sum(pos_f, axis=0, keepdims=True)

    @pl.when(step == _NB - 1)
    def _finish():
        lse12 = jnp.log(s1_ref[...]) + jnp.log(s2_ref[...])
        n = n_ref[...]
        ncl = jnp.maximum(n, 1.0)
        rows = (p_ref[...] - lse12 * n) / ncl
        loss_ref[...] = -jnp.sum(rows, axis=1, keepdims=True) / _B


def _gather_kernel(lab_ref, c_ref, qc_ref, sem):
    for i in range(_B):
        l = lab_ref[i]
        pltpu.make_async_copy(c_ref.at[pl.ds(l, 1), :],
                              qc_ref.at[pl.ds(i, 1), :], sem).start()
    for _ in range(_B):
        pltpu.make_async_copy(c_ref.at[pl.ds(0, 1), :],
                              qc_ref.at[pl.ds(0, 1), :], sem).wait()


def _scatter_kernel(lab_ref, win_ref, qe1_ref, qe2_ref, b1_ref, b2_ref,
                    o1_ref, o2_ref, sem1, sem2):
    del b1_ref, b2_ref
    for i in range(_B):
        l = lab_ref[i]
        w = win_ref[i]
        pltpu.make_async_copy(qe1_ref.at[pl.ds(w, 1)],
                              o1_ref.at[pl.ds(l, 1)], sem1).start()
        pltpu.make_async_copy(qe2_ref.at[pl.ds(w, 1)],
                              o2_ref.at[pl.ds(l, 1)], sem2).start()
    for _ in range(_B):
        pltpu.make_async_copy(qe1_ref.at[pl.ds(0, 1)],
                              o1_ref.at[pl.ds(0, 1)], sem1).wait()
        pltpu.make_async_copy(qe2_ref.at[pl.ds(0, 1)],
                              o2_ref.at[pl.ds(0, 1)], sem2).wait()


def kernel(mem1, mem2, query_emb1, query_emb2, coords, labels):
    labels = labels.astype(jnp.int32)

    qc3 = pl.pallas_call(
        _gather_kernel,
        grid_spec=pltpu.PrefetchScalarGridSpec(
            num_scalar_prefetch=1,
            grid=(1,),
            in_specs=[pl.BlockSpec(memory_space=pl.ANY)],
            out_specs=pl.BlockSpec((_B, 3), lambda i, lab: (0, 0)),
            scratch_shapes=[pltpu.SemaphoreType.DMA],
        ),
        out_shape=jax.ShapeDtypeStruct((_B, 3), jnp.float32),
        compiler_params=pltpu.CompilerParams(
            dimension_semantics=("arbitrary",)),
    )(labels, coords)
    qc2row = 0.5 * jnp.sum(qc3 * qc3, axis=1)[None, :]

    loss2d, o1, o2 = pl.pallas_call(
        _main_kernel,
        grid=(_NB,),
        in_specs=[
            pl.BlockSpec((_B, _D), lambda i: (0, 0)),
            pl.BlockSpec((_B, _D), lambda i: (0, 0)),
            pl.BlockSpec((_B, 3), lambda i: (0, 0)),
            pl.BlockSpec((1, _B), lambda i: (0, 0)),
            pl.BlockSpec((_BM, _D), lambda i: (i, 0)),
            pl.BlockSpec((_BM, _D), lambda i: (i, 0)),
            pl.BlockSpec((_BM, 3), lambda i: (i, 0)),
        ],
        out_specs=[
            pl.BlockSpec((1, 1), lambda i: (0, 0)),
            pl.BlockSpec((_BM, _D), lambda i: (i, 0)),
            pl.BlockSpec((_BM, _D), lambda i: (i, 0)),
        ],
        out_shape=[
            jax.ShapeDtypeStruct((1, 1), jnp.float32),
            jax.ShapeDtypeStruct((_M, _D), jnp.float32),
            jax.ShapeDtypeStruct((_M, _D), jnp.float32),
        ],
        scratch_shapes=[pltpu.VMEM((1, _B), jnp.float32) for _ in range(4)]
                       + [pltpu.VMEM((_B, _D), jnp.bfloat16) for _ in range(2)],
        compiler_params=pltpu.CompilerParams(
            dimension_semantics=("arbitrary",)),
    )(query_emb1, query_emb2, qc3, qc2row, mem1, mem2, coords)

    j = jnp.arange(_B, dtype=jnp.int32)
    eq = labels[:, None] == labels[None, :]
    winner = jnp.max(jnp.where(eq, j[None, :], -1), axis=1).astype(jnp.int32)

    grid_spec = pltpu.PrefetchScalarGridSpec(
        num_scalar_prefetch=2,
        grid=(1,),
        in_specs=[
            pl.BlockSpec(memory_space=pl.ANY),
            pl.BlockSpec(memory_space=pl.ANY),
            pl.BlockSpec(memory_space=pl.ANY),
            pl.BlockSpec(memory_space=pl.ANY),
        ],
        out_specs=[
            pl.BlockSpec(memory_space=pl.ANY),
            pl.BlockSpec(memory_space=pl.ANY),
        ],
        scratch_shapes=[pltpu.SemaphoreType.DMA, pltpu.SemaphoreType.DMA],
    )
    new_mem1, new_mem2 = pl.pallas_call(
        _scatter_kernel,
        grid_spec=grid_spec,
        out_shape=[
            jax.ShapeDtypeStruct((_M, _D), jnp.float32),
            jax.ShapeDtypeStruct((_M, _D), jnp.float32),
        ],
        input_output_aliases={4: 0, 5: 1},
        compiler_params=pltpu.CompilerParams(
            dimension_semantics=("arbitrary",)),
    )(labels, winner, query_emb1, query_emb2, o1, o2)

    return (loss2d[0, 0], new_mem1, new_mem2)
